# baseline (speedup 1.0000x reference)
import functools

import jax
import jax.numpy as jnp
from jax import lax
from jax.experimental import pallas as pl
from jax.experimental.pallas import tpu as pltpu
from jax.experimental.pallas import tpu_sc as plsc

_NC = 2
_NS = 16
_NW = _NC * _NS
_K = 400


def _sc_mesh():
    return plsc.VectorSubcoreMesh(
        core_axis_name="c", subcore_axis_name="s",
        num_cores=_NC, num_subcores=_NS)


def _project(x, wa_t, wb_t):
    n, c = x.shape

    def body(x_ref, wa_ref, wb_ref, px_ref, qx_ref):
        xv = x_ref[...]
        px_ref[...] = jnp.dot(xv, wa_ref[...], preferred_element_type=jnp.float32)
        qx_ref[...] = jnp.dot(xv, wb_ref[...], preferred_element_type=jnp.float32)

    return pl.pallas_call(
        body,
        out_shape=(jax.ShapeDtypeStruct((n, c), jnp.float32),
                   jax.ShapeDtypeStruct((n, c), jnp.float32)),
    )(x, wa_t, wb_t)


def _gather_add(px, qx, src, dst):
    n, c = px.shape
    e = src.shape[0]
    epw = e // _NW
    nch = epw // _K
    c16 = c // 16

    @functools.partial(
        pl.kernel,
        out_type=jax.ShapeDtypeStruct((e, c), jnp.float32),
        mesh=_sc_mesh(),
        scratch_types=[
            pltpu.VMEM((_K,), jnp.int32),
            pltpu.VMEM((_K,), jnp.int32),
            pltpu.VMEM((_K, c), jnp.float32),
            pltpu.VMEM((_K, c), jnp.float32),
            pltpu.SemaphoreType.DMA,
            pltpu.SemaphoreType.DMA,
        ],
    )
    def run(px_hbm, qx_hbm, src_hbm, dst_hbm, g_hbm,
            idx_s, idx_d, rows_s, rows_d, sem_s, sem_d):
        wid = lax.axis_index("s") * _NC + lax.axis_index("c")
        base0 = wid * epw

        def chunk(j, carry):
            base = base0 + j * _K
            pltpu.sync_copy(src_hbm.at[pl.ds(base, _K)], idx_s)
            pltpu.sync_copy(dst_hbm.at[pl.ds(base, _K)], idx_d)
            cp_s = pltpu.async_copy(px_hbm.at[idx_s], rows_s, sem_s)
            cp_d = pltpu.async_copy(qx_hbm.at[idx_d], rows_d, sem_d)
            cp_s.wait()
            cp_d.wait()

            def add_row(i, carry2):
                for t in range(c16):
                    sl = pl.ds(t * 16, 16)
                    rows_s[i, sl] = rows_s[i, sl] + rows_d[i, sl]
                return carry2

            lax.fori_loop(0, _K, add_row, 0)
            pltpu.sync_copy(rows_s, g_hbm.at[pl.ds(base, _K)])
            return carry

        lax.fori_loop(0, nch, chunk, 0)

    return run(px, qx, src, dst)


def _edge_mlp(g, ea, half, c0_t, b0, w1_t, b1, w2_t, b2, w3_t, b3, lnw, lnb):
    e, c = ea.shape
    blk = 2000
    grid = e // (2 * blk)
    off = half * grid

    def body(g_ref, ea_ref, c0_ref, b0_ref, w1_ref, b1_ref, w2_ref, b2_ref,
             w3_ref, b3_ref, lnw_ref, lnb_ref, e_ref):
        ea_v = ea_ref[...]
        h = (g_ref[...] + b0_ref[...]
             + jnp.dot(ea_v.astype(jnp.bfloat16), c0_ref[...],
                       preferred_element_type=jnp.float32))
        h = jnp.maximum(h, 0.0).astype(jnp.bfloat16)
        h = jnp.dot(h, w1_ref[...], preferred_element_type=jnp.float32) + b1_ref[...]
        h = jnp.maximum(h, 0.0).astype(jnp.bfloat16)
        h = jnp.dot(h, w2_ref[...], preferred_element_type=jnp.float32) + b2_ref[...]
        h = jnp.maximum(h, 0.0).astype(jnp.bfloat16)
        h = jnp.dot(h, w3_ref[...], preferred_element_type=jnp.float32) + b3_ref[...]
        mu = jnp.mean(h, axis=-1, keepdims=True)
        hc = h - mu
        var = jnp.mean(hc * hc, axis=-1, keepdims=True)
        hn = hc * lax.rsqrt(var + 1e-5)
        e_ref[...] = ea_v + hn * lnw_ref[...] + lnb_ref[...]

    full = pl.BlockSpec((c, c), lambda i: (0, 0))
    vec = pl.BlockSpec((1, c), lambda i: (0, 0))
    ibs = pl.BlockSpec((blk, c), lambda i: (i + off, 0))
    obs = pl.BlockSpec((blk, c), lambda i: (i, 0))
    return pl.pallas_call(
        body,
        grid=(grid,),
        in_specs=[ibs, ibs, full, vec, full, vec, full, vec, full, vec, vec, vec],
        out_specs=obs,
        out_shape=jax.ShapeDtypeStruct((e // 2, c), jnp.float32),
    )(g, ea, c0_t, b0, w1_t, b1, w2_t, b2, w3_t, b3, lnw, lnb)


def _segment_sum(e_rows, dst, zeros, n):
    e, c = e_rows.shape
    kk = 128
    ept = e // _NW
    nch = ept // kk
    tail = ept - nch * kk
    rpt = (n // _NS) // 8 * 8
    rtail = n - rpt * _NS

    @functools.partial(
        pl.kernel,
        out_type=jax.ShapeDtypeStruct((_NC * n, c), jnp.float32),
        mesh=_sc_mesh(),
        scratch_types=[
            pltpu.VMEM((kk,), jnp.int32),
            pltpu.VMEM((kk, c), jnp.float32),
            pltpu.VMEM((tail,), jnp.int32) if tail else None,
            pltpu.VMEM((tail, c), jnp.float32) if tail else None,
            pltpu.VMEM_SHARED((n, c), jnp.float32),
        ],
    )
    def run(e_hbm, dst_hbm, z_hbm, out_hbm, idx_v, upd_v, idx_t, upd_t, acc):
        ci = lax.axis_index("c")
        s = lax.axis_index("s")
        pltpu.sync_copy(z_hbm.at[pl.ds(s * rpt, rpt)], acc.at[pl.ds(s * rpt, rpt)])
        if rtail:
            @pl.when(s == 0)
            def _():
                pltpu.sync_copy(z_hbm.at[pl.ds(rpt * _NS, rtail)],
                                acc.at[pl.ds(rpt * _NS, rtail)])
        plsc.subcore_barrier()

        base0 = (ci * _NS + s) * ept

        def chunk(j, carry):
            base = base0 + j * kk
            pltpu.sync_copy(dst_hbm.at[pl.ds(base, kk)], idx_v)
            pltpu.sync_copy(e_hbm.at[pl.ds(base, kk)], upd_v)
            pltpu.sync_copy(upd_v, acc.at[idx_v], add=True)
            return carry

        lax.fori_loop(0, nch, chunk, 0)
        if tail:
            base = base0 + nch * kk
            pltpu.sync_copy(dst_hbm.at[pl.ds(base, tail)], idx_t)
            pltpu.sync_copy(e_hbm.at[pl.ds(base, tail)], upd_t)
            pltpu.sync_copy(upd_t, acc.at[idx_t], add=True)
        plsc.subcore_barrier()
        pltpu.sync_copy(acc.at[pl.ds(s * rpt, rpt)],
                        out_hbm.at[pl.ds(ci * n + s * rpt, rpt)])
        if rtail:
            @pl.when(s == 0)
            def _():
                pltpu.sync_copy(acc.at[pl.ds(rpt * _NS, rtail)],
                                out_hbm.at[pl.ds(ci * n + rpt * _NS, rtail)])

    return run(e_rows, dst, zeros).reshape(_NC, n, c)


def _node_mlp(x, partials0, partials1, va_t, vb_t, b0, w1_t, b1, w2_t, b2,
              w3_t, b3, lnw, lnb):
    n, c = x.shape
    blk = 2000
    grid = n // blk

    def body(x_ref, p_ref, q_ref, va_ref, vb_ref, b0_ref, w1_ref, b1_ref,
             w2_ref, b2_ref, w3_ref, b3_ref, lnw_ref, lnb_ref, o_ref):
        xv = x_ref[...]
        agg = (p_ref[0] + p_ref[1]) + (q_ref[0] + q_ref[1])
        h = (jnp.dot(xv, va_ref[...], preferred_element_type=jnp.float32)
             + jnp.dot(agg, vb_ref[...], preferred_element_type=jnp.float32)
             + b0_ref[...])
        h = jnp.maximum(h, 0.0)
        h = jnp.dot(h, w1_ref[...], preferred_element_type=jnp.float32) + b1_ref[...]
        h = jnp.maximum(h, 0.0)
        h = jnp.dot(h, w2_ref[...], preferred_element_type=jnp.float32) + b2_ref[...]
        h = jnp.maximum(h, 0.0)
        h = jnp.dot(h, w3_ref[...], preferred_element_type=jnp.float32) + b3_ref[...]
        mu = jnp.mean(h, axis=-1, keepdims=True)
        hc = h - mu
        var = jnp.mean(hc * hc, axis=-1, keepdims=True)
        hn = hc * lax.rsqrt(var + 1e-5)
        o_ref[...] = xv + hn * lnw_ref[...] + lnb_ref[...]

    full = pl.BlockSpec((c, c), lambda i: (0, 0))
    vec = pl.BlockSpec((1, c), lambda i: (0, 0))
    nbs = pl.BlockSpec((blk, c), lambda i: (i, 0))
    pbs = pl.BlockSpec((_NC, blk, c), lambda i: (0, i, 0))
    return pl.pallas_call(
        body,
        grid=(grid,),
        in_specs=[nbs, pbs, pbs, full, full, vec, full, vec, full, vec, full,
                  vec, vec, vec],
        out_specs=nbs,
        out_shape=jax.ShapeDtypeStruct((n, c), jnp.float32),
    )(x, partials0, partials1, va_t, vb_t, b0, w1_t, b1, w2_t, b2, w3_t, b3,
      lnw, lnb)


def kernel(x, edge_attr, edge_index, params):
    n, c = x.shape
    ep = params["edge"]
    np_ = params["node"]
    w0 = ep["W"][0]
    wa_t = w0[:, :c].T
    wb_t = w0[:, c:2 * c].T
    c0_t = w0[:, 2 * c:].T
    v0 = np_["W"][0]
    va_t = v0[:, :c].T
    vb_t = v0[:, c:].T

    def row(v):
        return v.reshape(1, c)

    src = edge_index[0]
    dst = edge_index[1]

    px, qx = _project(x, wa_t, wb_t)
    bf = jnp.bfloat16
    edge_w = (c0_t.astype(bf), row(ep["b"][0]),
              ep["W"][1].T.astype(bf), row(ep["b"][1]),
              ep["W"][2].T.astype(bf), row(ep["b"][2]),
              ep["W"][3].T.astype(bf), row(ep["b"][3]),
              row(ep["ln_w"]), row(ep["ln_b"]))
    zeros = jnp.zeros((n, c), jnp.float32)
    eh = src.shape[0] // 2

    g = _gather_add(px, qx, src, dst)
    e0 = _edge_mlp(g, edge_attr, 0, *edge_w)
    e1 = _edge_mlp(g, edge_attr, 1, *edge_w)
    p0 = _segment_sum(e0, dst[:eh], zeros, n)
    p1 = _segment_sum(e1, dst[eh:], zeros, n)
    e = jnp.concatenate([e0, e1], axis=0)
    x_out = _node_mlp(x, p0, p1, va_t, vb_t, row(np_["b"][0]), np_["W"][1].T,
                      row(np_["b"][1]), np_["W"][2].T, row(np_["b"][2]),
                      np_["W"][3].T, row(np_["b"][3]),
                      row(np_["ln_w"]), row(np_["ln_b"]))
    return (x_out, e)


# speedup vs baseline: 1.1498x; 1.1498x over previous
import functools

import jax
import jax.numpy as jnp
from jax import lax
from jax.experimental import pallas as pl
from jax.experimental.pallas import tpu as pltpu
from jax.experimental.pallas import tpu_sc as plsc

_NC = 2
_NS = 16
_NW = _NC * _NS
_K = 400


def _sc_mesh():
    return plsc.VectorSubcoreMesh(
        core_axis_name="c", subcore_axis_name="s",
        num_cores=_NC, num_subcores=_NS)


def _project(x, wa_t, wb_t):
    n, c = x.shape

    def body(x_ref, wa_ref, wb_ref, px_ref, qx_ref):
        xv = x_ref[...]
        px_ref[...] = jnp.dot(xv, wa_ref[...], preferred_element_type=jnp.float32)
        qx_ref[...] = jnp.dot(xv, wb_ref[...], preferred_element_type=jnp.float32)

    return pl.pallas_call(
        body,
        out_shape=(jax.ShapeDtypeStruct((n, c), jnp.float32),
                   jax.ShapeDtypeStruct((n, c), jnp.float32)),
    )(x, wa_t, wb_t)


def _gather_add(px, qx, src, dst):
    n, c = px.shape
    e = src.shape[0]
    epw = e // _NW
    nch = epw // _K
    c16 = c // 16

    @functools.partial(
        pl.kernel,
        out_type=jax.ShapeDtypeStruct((e, c), jnp.float32),
        mesh=_sc_mesh(),
        scratch_types=[
            pltpu.VMEM((_K,), jnp.int32),
            pltpu.VMEM((_K,), jnp.int32),
            pltpu.VMEM((_K, c), jnp.float32),
            pltpu.VMEM((_K, c), jnp.float32),
            pltpu.SemaphoreType.DMA,
            pltpu.SemaphoreType.DMA,
        ],
    )
    def run(px_hbm, qx_hbm, src_hbm, dst_hbm, g_hbm,
            idx_s, idx_d, rows_s, rows_d, sem_s, sem_d):
        wid = lax.axis_index("s") * _NC + lax.axis_index("c")
        base0 = wid * epw

        def chunk(j, carry):
            base = base0 + j * _K
            pltpu.sync_copy(src_hbm.at[pl.ds(base, _K)], idx_s)
            pltpu.sync_copy(dst_hbm.at[pl.ds(base, _K)], idx_d)
            cp_s = pltpu.async_copy(px_hbm.at[idx_s], rows_s, sem_s)
            cp_d = pltpu.async_copy(qx_hbm.at[idx_d], rows_d, sem_d)
            cp_s.wait()
            cp_d.wait()

            def add_row(i, carry2):
                for t in range(c16):
                    sl = pl.ds(t * 16, 16)
                    rows_s[i, sl] = rows_s[i, sl] + rows_d[i, sl]
                return carry2

            lax.fori_loop(0, _K, add_row, 0)
            pltpu.sync_copy(rows_s, g_hbm.at[pl.ds(base, _K)])
            return carry

        lax.fori_loop(0, nch, chunk, 0)

    return run(px, qx, src, dst)


def _edge_mlp(g, ea, c0_t, b0, w1_t, b1, w2_t, b2, w3_t, b3, lnw, lnb):
    e, c = ea.shape
    blk = 4000
    grid = e // blk

    def body(g_ref, ea_ref, c0_ref, b0_ref, w1_ref, b1_ref, w2_ref, b2_ref,
             w3_ref, b3_ref, lnw_ref, lnb_ref, e_ref):
        ea_v = ea_ref[...]
        h = (g_ref[...] + b0_ref[...]
             + jnp.dot(ea_v.astype(jnp.bfloat16), c0_ref[...],
                       preferred_element_type=jnp.float32))
        h = jnp.maximum(h, 0.0).astype(jnp.bfloat16)
        h = jnp.dot(h, w1_ref[...], preferred_element_type=jnp.float32) + b1_ref[...]
        h = jnp.maximum(h, 0.0).astype(jnp.bfloat16)
        h = jnp.dot(h, w2_ref[...], preferred_element_type=jnp.float32) + b2_ref[...]
        h = jnp.maximum(h, 0.0).astype(jnp.bfloat16)
        h = jnp.dot(h, w3_ref[...], preferred_element_type=jnp.float32) + b3_ref[...]
        mu = jnp.mean(h, axis=-1, keepdims=True)
        hc = h - mu
        var = jnp.mean(hc * hc, axis=-1, keepdims=True)
        hn = hc * lax.rsqrt(var + 1e-5)
        e_ref[...] = ea_v + hn * lnw_ref[...] + lnb_ref[...]

    full = pl.BlockSpec((c, c), lambda i: (0, 0))
    vec = pl.BlockSpec((1, c), lambda i: (0, 0))
    ebs = pl.BlockSpec((blk, c), lambda i: (i, 0))
    return pl.pallas_call(
        body,
        grid=(grid,),
        in_specs=[ebs, ebs, full, vec, full, vec, full, vec, full, vec, vec, vec],
        out_specs=ebs,
        out_shape=jax.ShapeDtypeStruct((e, c), jnp.float32),
    )(g, ea, c0_t, b0, w1_t, b1, w2_t, b2, w3_t, b3, lnw, lnb)


def _segment_sum(e_rows, dst, zeros, n):
    e, c = e_rows.shape
    kk = 128
    ept = e // _NW
    nch = ept // kk
    tail = ept - nch * kk
    rpt = (n // _NS) // 8 * 8
    rtail = n - rpt * _NS

    @functools.partial(
        pl.kernel,
        out_type=jax.ShapeDtypeStruct((_NC * n, c), jnp.float32),
        mesh=_sc_mesh(),
        scratch_types=[
            pltpu.VMEM((2, kk), jnp.int32),
            pltpu.VMEM((kk, c), jnp.float32),
            pltpu.VMEM((kk, c), jnp.float32),
            pltpu.VMEM((tail,), jnp.int32) if tail else None,
            pltpu.VMEM((tail, c), jnp.float32) if tail else None,
            pltpu.VMEM_SHARED((n, c), jnp.float32),
            pltpu.SemaphoreType.DMA,
            pltpu.SemaphoreType.DMA,
            pltpu.SemaphoreType.DMA,
            pltpu.SemaphoreType.DMA,
        ],
    )
    def run(e_hbm, dst_hbm, z_hbm, out_hbm, idx2, upd_a, upd_b, idx_t, upd_t,
            acc, sem_i, sem_ua, sem_ub, sem_s):
        ci = lax.axis_index("c")
        s = lax.axis_index("s")
        pltpu.sync_copy(z_hbm.at[pl.ds(s * rpt, rpt)], acc.at[pl.ds(s * rpt, rpt)])
        if rtail:
            @pl.when(s == 0)
            def _():
                pltpu.sync_copy(z_hbm.at[pl.ds(rpt * _NS, rtail)],
                                acc.at[pl.ds(rpt * _NS, rtail)])
        plsc.subcore_barrier()

        base0 = (ci * _NS + s) * ept
        nstep = nch // 2

        pltpu.sync_copy(dst_hbm.at[pl.ds(base0, kk)], idx2.at[0])
        pltpu.async_copy(e_hbm.at[pl.ds(base0, kk)], upd_a, sem_ua).wait()

        def pair(jj, carry):
            j0 = 2 * jj
            b1 = base0 + (j0 + 1) * kk
            b2 = base0 + (j0 + 2) * kk
            pltpu.async_copy(dst_hbm.at[pl.ds(b1, kk)], idx2.at[1], sem_i).wait()
            cp_b = pltpu.async_copy(e_hbm.at[pl.ds(b1, kk)], upd_b, sem_ub)
            pltpu.async_copy(upd_a, acc.at[idx2.at[0]], sem_s, add=True).wait()
            cp_b.wait()

            @pl.when(jj + 1 < nstep)
            def _():
                pltpu.async_copy(dst_hbm.at[pl.ds(b2, kk)], idx2.at[0],
                                 sem_i).wait()
                pltpu.async_copy(e_hbm.at[pl.ds(b2, kk)], upd_a, sem_ua)
            pltpu.async_copy(upd_b, acc.at[idx2.at[1]], sem_s, add=True).wait()

            @pl.when(jj + 1 < nstep)
            def _():
                pltpu.make_async_copy(e_hbm.at[pl.ds(b2, kk)], upd_a,
                                      sem_ua).wait()
            return carry

        lax.fori_loop(0, nstep, pair, 0)
        if tail:
            base = base0 + nch * kk
            pltpu.sync_copy(dst_hbm.at[pl.ds(base, tail)], idx_t)
            pltpu.sync_copy(e_hbm.at[pl.ds(base, tail)], upd_t)
            pltpu.sync_copy(upd_t, acc.at[idx_t], add=True)
        plsc.subcore_barrier()
        pltpu.sync_copy(acc.at[pl.ds(s * rpt, rpt)],
                        out_hbm.at[pl.ds(ci * n + s * rpt, rpt)])
        if rtail:
            @pl.when(s == 0)
            def _():
                pltpu.sync_copy(acc.at[pl.ds(rpt * _NS, rtail)],
                                out_hbm.at[pl.ds(ci * n + rpt * _NS, rtail)])

    return run(e_rows, dst, zeros).reshape(_NC, n, c)


def _node_mlp(x, partials, va_t, vb_t, b0, w1_t, b1, w2_t, b2,
              w3_t, b3, lnw, lnb):
    n, c = x.shape
    blk = 2000
    grid = n // blk

    def body(x_ref, p_ref, va_ref, vb_ref, b0_ref, w1_ref, b1_ref,
             w2_ref, b2_ref, w3_ref, b3_ref, lnw_ref, lnb_ref, o_ref):
        xv = x_ref[...]
        agg = p_ref[0] + p_ref[1]
        h = (jnp.dot(xv, va_ref[...], preferred_element_type=jnp.float32)
             + jnp.dot(agg, vb_ref[...], preferred_element_type=jnp.float32)
             + b0_ref[...])
        h = jnp.maximum(h, 0.0)
        h = jnp.dot(h, w1_ref[...], preferred_element_type=jnp.float32) + b1_ref[...]
        h = jnp.maximum(h, 0.0)
        h = jnp.dot(h, w2_ref[...], preferred_element_type=jnp.float32) + b2_ref[...]
        h = jnp.maximum(h, 0.0)
        h = jnp.dot(h, w3_ref[...], preferred_element_type=jnp.float32) + b3_ref[...]
        mu = jnp.mean(h, axis=-1, keepdims=True)
        hc = h - mu
        var = jnp.mean(hc * hc, axis=-1, keepdims=True)
        hn = hc * lax.rsqrt(var + 1e-5)
        o_ref[...] = xv + hn * lnw_ref[...] + lnb_ref[...]

    full = pl.BlockSpec((c, c), lambda i: (0, 0))
    vec = pl.BlockSpec((1, c), lambda i: (0, 0))
    nbs = pl.BlockSpec((blk, c), lambda i: (i, 0))
    pbs = pl.BlockSpec((_NC, blk, c), lambda i: (0, i, 0))
    return pl.pallas_call(
        body,
        grid=(grid,),
        in_specs=[nbs, pbs, full, full, vec, full, vec, full, vec, full,
                  vec, vec, vec],
        out_specs=nbs,
        out_shape=jax.ShapeDtypeStruct((n, c), jnp.float32),
    )(x, partials, va_t, vb_t, b0, w1_t, b1, w2_t, b2, w3_t, b3,
      lnw, lnb)


def kernel(x, edge_attr, edge_index, params):
    n, c = x.shape
    ep = params["edge"]
    np_ = params["node"]
    w0 = ep["W"][0]
    wa_t = w0[:, :c].T
    wb_t = w0[:, c:2 * c].T
    c0_t = w0[:, 2 * c:].T
    v0 = np_["W"][0]
    va_t = v0[:, :c].T
    vb_t = v0[:, c:].T

    def row(v):
        return v.reshape(1, c)

    src = edge_index[0]
    dst = edge_index[1]

    px, qx = _project(x, wa_t, wb_t)
    bf = jnp.bfloat16
    edge_w = (c0_t.astype(bf), row(ep["b"][0]),
              ep["W"][1].T.astype(bf), row(ep["b"][1]),
              ep["W"][2].T.astype(bf), row(ep["b"][2]),
              ep["W"][3].T.astype(bf), row(ep["b"][3]),
              row(ep["ln_w"]), row(ep["ln_b"]))
    zeros = jnp.zeros((n, c), jnp.float32)
    eh = src.shape[0] // 2

    g = _gather_add(px, qx, src, dst)
    e = _edge_mlp(g, edge_attr, *edge_w)
    partials = _segment_sum(e, dst, zeros, n)
    x_out = _node_mlp(x, partials, va_t, vb_t, row(np_["b"][0]), np_["W"][1].T,
                      row(np_["b"][1]), np_["W"][2].T, row(np_["b"][2]),
                      np_["W"][3].T, row(np_["b"][3]),
                      row(np_["ln_w"]), row(np_["ln_b"]))
    return (x_out, e)


# speedup vs baseline: 1.3464x; 1.1709x over previous
import functools

import jax
import jax.numpy as jnp
from jax import lax
from jax.experimental import pallas as pl
from jax.experimental.pallas import tpu as pltpu
from jax.experimental.pallas import tpu_sc as plsc

_NC = 2
_NS = 16
_NW = _NC * _NS
_K = 400


def _sc_mesh():
    return plsc.VectorSubcoreMesh(
        core_axis_name="c", subcore_axis_name="s",
        num_cores=_NC, num_subcores=_NS)


def _project(x, wa_t, wb_t):
    n, c = x.shape

    def body(x_ref, wa_ref, wb_ref, px_ref, qx_ref):
        xv = x_ref[...]
        px_ref[...] = jnp.dot(xv, wa_ref[...], preferred_element_type=jnp.float32)
        qx_ref[...] = jnp.dot(xv, wb_ref[...], preferred_element_type=jnp.float32)

    return pl.pallas_call(
        body,
        out_shape=(jax.ShapeDtypeStruct((n, c), jnp.float32),
                   jax.ShapeDtypeStruct((n, c), jnp.float32)),
    )(x, wa_t, wb_t)


def _gather_add(px, qx, src, dst):
    n, c = px.shape
    e = src.shape[0]
    kk = 200
    epw = e // _NW
    nch = epw // kk
    assert nch % 2 == 0
    nstep = nch // 2
    c16 = c // 16

    @functools.partial(
        pl.kernel,
        out_type=jax.ShapeDtypeStruct((e, c), jnp.float32),
        mesh=_sc_mesh(),
        scratch_types=[
            pltpu.VMEM((kk,), jnp.int32),
            pltpu.VMEM((kk,), jnp.int32),
            pltpu.VMEM((kk,), jnp.int32),
            pltpu.VMEM((kk,), jnp.int32),
            pltpu.VMEM((kk, c), jnp.float32),
            pltpu.VMEM((kk, c), jnp.float32),
            pltpu.VMEM((kk, c), jnp.float32),
            pltpu.VMEM((kk, c), jnp.float32),
            pltpu.SemaphoreType.DMA,
            pltpu.SemaphoreType.DMA,
            pltpu.SemaphoreType.DMA,
            pltpu.SemaphoreType.DMA,
            pltpu.SemaphoreType.DMA,
        ],
    )
    def run(px_hbm, qx_hbm, src_hbm, dst_hbm, g_hbm,
            idx_s0, idx_s1, idx_d0, idx_d1, p0, q0, p1, q1,
            sem_a, sem_b, sem_i, sem_w0, sem_w1):
        idx_s = (idx_s0, idx_s1)
        idx_d = (idx_d0, idx_d1)
        wid = lax.axis_index("s") * _NC + lax.axis_index("c")
        base0 = wid * epw

        def add_rows(pr, qr):
            def add_row(i, carry2):
                for t in range(c16):
                    sl = pl.ds(t * 16, 16)
                    pr[i, sl] = pr[i, sl] + qr[i, sl]
                return carry2
            lax.fori_loop(0, kk, add_row, 0)

        def load_idx(j, slot):
            base = base0 + j * kk
            pltpu.async_copy(src_hbm.at[pl.ds(base, kk)], idx_s[slot],
                             sem_i).wait()
            pltpu.async_copy(dst_hbm.at[pl.ds(base, kk)], idx_d[slot],
                             sem_i).wait()

        def start_gather(slot, pr, qr):
            sem = sem_a if slot == 0 else sem_b
            pltpu.async_copy(px_hbm.at[idx_s[slot]], pr, sem)
            pltpu.async_copy(qx_hbm.at[idx_d[slot]], qr, sem)

        def wait_gather(slot, pr, qr):
            sem = sem_a if slot == 0 else sem_b
            pltpu.make_async_copy(px_hbm.at[idx_s[slot]], pr, sem).wait()
            pltpu.make_async_copy(qx_hbm.at[idx_d[slot]], qr, sem).wait()

        load_idx(0, 0)
        start_gather(0, p0, q0)

        def pair(jj, carry):
            j0 = 2 * jj
            j1 = j0 + 1
            j2 = j0 + 2
            g_j0 = g_hbm.at[pl.ds(base0 + j0 * kk, kk)]
            g_j1 = g_hbm.at[pl.ds(base0 + j1 * kk, kk)]

            @pl.when(jj > 0)
            def _():
                pltpu.make_async_copy(
                    p1, g_hbm.at[pl.ds(base0 + (j1 - 2) * kk, kk)],
                    sem_w1).wait()
            load_idx(j1, 1)
            start_gather(1, p1, q1)

            wait_gather(0, p0, q0)
            add_rows(p0, q0)
            pltpu.async_copy(p0, g_j0, sem_w0)

            @pl.when(jj + 1 < nstep)
            def _():
                load_idx(j2, 0)
                pltpu.make_async_copy(p0, g_j0, sem_w0).wait()
                start_gather(0, p0, q0)

            wait_gather(1, p1, q1)
            add_rows(p1, q1)
            pltpu.async_copy(p1, g_j1, sem_w1)

            @pl.when(jj + 1 == nstep)
            def _():
                pltpu.make_async_copy(p0, g_j0, sem_w0).wait()
                pltpu.make_async_copy(p1, g_j1, sem_w1).wait()
            return carry

        lax.fori_loop(0, nstep, pair, 0)

    return run(px, qx, src, dst)


def _edge_mlp(g, ea, c0_t, b0, w1_t, b1, w2_t, b2, w3_t, b3, lnw, lnb):
    e, c = ea.shape
    blk = 4000
    grid = e // blk

    def body(g_ref, ea_ref, c0_ref, b0_ref, w1_ref, b1_ref, w2_ref, b2_ref,
             w3_ref, b3_ref, lnw_ref, lnb_ref, e_ref):
        ea_v = ea_ref[...]
        h = (g_ref[...] + b0_ref[...]
             + jnp.dot(ea_v.astype(jnp.bfloat16), c0_ref[...],
                       preferred_element_type=jnp.float32))
        h = jnp.maximum(h, 0.0).astype(jnp.bfloat16)
        h = jnp.dot(h, w1_ref[...], preferred_element_type=jnp.float32) + b1_ref[...]
        h = jnp.maximum(h, 0.0).astype(jnp.bfloat16)
        h = jnp.dot(h, w2_ref[...], preferred_element_type=jnp.float32) + b2_ref[...]
        h = jnp.maximum(h, 0.0).astype(jnp.bfloat16)
        h = jnp.dot(h, w3_ref[...], preferred_element_type=jnp.float32) + b3_ref[...]
        mu = jnp.mean(h, axis=-1, keepdims=True)
        hc = h - mu
        var = jnp.mean(hc * hc, axis=-1, keepdims=True)
        hn = hc * lax.rsqrt(var + 1e-5)
        e_ref[...] = ea_v + hn * lnw_ref[...] + lnb_ref[...]

    full = pl.BlockSpec((c, c), lambda i: (0, 0))
    vec = pl.BlockSpec((1, c), lambda i: (0, 0))
    ebs = pl.BlockSpec((blk, c), lambda i: (i, 0))
    return pl.pallas_call(
        body,
        grid=(grid,),
        in_specs=[ebs, ebs, full, vec, full, vec, full, vec, full, vec, vec, vec],
        out_specs=ebs,
        out_shape=jax.ShapeDtypeStruct((e, c), jnp.float32),
    )(g, ea, c0_t, b0, w1_t, b1, w2_t, b2, w3_t, b3, lnw, lnb)


def _segment_sum(e_rows, dst, zeros, n):
    e, c = e_rows.shape
    kk = 128
    ept = e // _NW
    nch = ept // kk
    tail = ept - nch * kk
    rpt = (n // _NS) // 8 * 8
    rtail = n - rpt * _NS

    @functools.partial(
        pl.kernel,
        out_type=jax.ShapeDtypeStruct((_NC * n, c), jnp.float32),
        mesh=_sc_mesh(),
        scratch_types=[
            pltpu.VMEM((2, kk), jnp.int32),
            pltpu.VMEM((kk, c), jnp.float32),
            pltpu.VMEM((kk, c), jnp.float32),
            pltpu.VMEM((tail,), jnp.int32) if tail else None,
            pltpu.VMEM((tail, c), jnp.float32) if tail else None,
            pltpu.VMEM_SHARED((n, c), jnp.float32),
            pltpu.SemaphoreType.DMA,
            pltpu.SemaphoreType.DMA,
            pltpu.SemaphoreType.DMA,
            pltpu.SemaphoreType.DMA,
        ],
    )
    def run(e_hbm, dst_hbm, z_hbm, out_hbm, idx2, upd_a, upd_b, idx_t, upd_t,
            acc, sem_i, sem_ua, sem_ub, sem_s):
        ci = lax.axis_index("c")
        s = lax.axis_index("s")
        pltpu.sync_copy(z_hbm.at[pl.ds(s * rpt, rpt)], acc.at[pl.ds(s * rpt, rpt)])
        if rtail:
            @pl.when(s == 0)
            def _():
                pltpu.sync_copy(z_hbm.at[pl.ds(rpt * _NS, rtail)],
                                acc.at[pl.ds(rpt * _NS, rtail)])
        plsc.subcore_barrier()

        base0 = (ci * _NS + s) * ept
        nstep = nch // 2

        pltpu.sync_copy(dst_hbm.at[pl.ds(base0, kk)], idx2.at[0])
        pltpu.async_copy(e_hbm.at[pl.ds(base0, kk)], upd_a, sem_ua).wait()

        def pair(jj, carry):
            j0 = 2 * jj
            b1 = base0 + (j0 + 1) * kk
            b2 = base0 + (j0 + 2) * kk
            pltpu.async_copy(dst_hbm.at[pl.ds(b1, kk)], idx2.at[1], sem_i).wait()
            cp_b = pltpu.async_copy(e_hbm.at[pl.ds(b1, kk)], upd_b, sem_ub)
            pltpu.async_copy(upd_a, acc.at[idx2.at[0]], sem_s, add=True).wait()
            cp_b.wait()

            @pl.when(jj + 1 < nstep)
            def _():
                pltpu.async_copy(dst_hbm.at[pl.ds(b2, kk)], idx2.at[0],
                                 sem_i).wait()
                pltpu.async_copy(e_hbm.at[pl.ds(b2, kk)], upd_a, sem_ua)
            pltpu.async_copy(upd_b, acc.at[idx2.at[1]], sem_s, add=True).wait()

            @pl.when(jj + 1 < nstep)
            def _():
                pltpu.make_async_copy(e_hbm.at[pl.ds(b2, kk)], upd_a,
                                      sem_ua).wait()
            return carry

        lax.fori_loop(0, nstep, pair, 0)
        if tail:
            base = base0 + nch * kk
            pltpu.sync_copy(dst_hbm.at[pl.ds(base, tail)], idx_t)
            pltpu.sync_copy(e_hbm.at[pl.ds(base, tail)], upd_t)
            pltpu.sync_copy(upd_t, acc.at[idx_t], add=True)
        plsc.subcore_barrier()
        pltpu.sync_copy(acc.at[pl.ds(s * rpt, rpt)],
                        out_hbm.at[pl.ds(ci * n + s * rpt, rpt)])
        if rtail:
            @pl.when(s == 0)
            def _():
                pltpu.sync_copy(acc.at[pl.ds(rpt * _NS, rtail)],
                                out_hbm.at[pl.ds(ci * n + rpt * _NS, rtail)])

    return run(e_rows, dst, zeros).reshape(_NC, n, c)


def _node_mlp(x, partials, va_t, vb_t, b0, w1_t, b1, w2_t, b2,
              w3_t, b3, lnw, lnb):
    n, c = x.shape
    blk = 2000
    grid = n // blk

    def body(x_ref, p_ref, va_ref, vb_ref, b0_ref, w1_ref, b1_ref,
             w2_ref, b2_ref, w3_ref, b3_ref, lnw_ref, lnb_ref, o_ref):
        xv = x_ref[...]
        agg = p_ref[0] + p_ref[1]
        h = (jnp.dot(xv, va_ref[...], preferred_element_type=jnp.float32)
             + jnp.dot(agg, vb_ref[...], preferred_element_type=jnp.float32)
             + b0_ref[...])
        h = jnp.maximum(h, 0.0)
        h = jnp.dot(h, w1_ref[...], preferred_element_type=jnp.float32) + b1_ref[...]
        h = jnp.maximum(h, 0.0)
        h = jnp.dot(h, w2_ref[...], preferred_element_type=jnp.float32) + b2_ref[...]
        h = jnp.maximum(h, 0.0)
        h = jnp.dot(h, w3_ref[...], preferred_element_type=jnp.float32) + b3_ref[...]
        mu = jnp.mean(h, axis=-1, keepdims=True)
        hc = h - mu
        var = jnp.mean(hc * hc, axis=-1, keepdims=True)
        hn = hc * lax.rsqrt(var + 1e-5)
        o_ref[...] = xv + hn * lnw_ref[...] + lnb_ref[...]

    full = pl.BlockSpec((c, c), lambda i: (0, 0))
    vec = pl.BlockSpec((1, c), lambda i: (0, 0))
    nbs = pl.BlockSpec((blk, c), lambda i: (i, 0))
    pbs = pl.BlockSpec((_NC, blk, c), lambda i: (0, i, 0))
    return pl.pallas_call(
        body,
        grid=(grid,),
        in_specs=[nbs, pbs, full, full, vec, full, vec, full, vec, full,
                  vec, vec, vec],
        out_specs=nbs,
        out_shape=jax.ShapeDtypeStruct((n, c), jnp.float32),
    )(x, partials, va_t, vb_t, b0, w1_t, b1, w2_t, b2, w3_t, b3,
      lnw, lnb)


def kernel(x, edge_attr, edge_index, params):
    n, c = x.shape
    ep = params["edge"]
    np_ = params["node"]
    w0 = ep["W"][0]
    wa_t = w0[:, :c].T
    wb_t = w0[:, c:2 * c].T
    c0_t = w0[:, 2 * c:].T
    v0 = np_["W"][0]
    va_t = v0[:, :c].T
    vb_t = v0[:, c:].T

    def row(v):
        return v.reshape(1, c)

    src = edge_index[0]
    dst = edge_index[1]

    px, qx = _project(x, wa_t, wb_t)
    bf = jnp.bfloat16
    edge_w = (c0_t.astype(bf), row(ep["b"][0]),
              ep["W"][1].T.astype(bf), row(ep["b"][1]),
              ep["W"][2].T.astype(bf), row(ep["b"][2]),
              ep["W"][3].T.astype(bf), row(ep["b"][3]),
              row(ep["ln_w"]), row(ep["ln_b"]))
    zeros = jnp.zeros((n, c), jnp.float32)
    eh = src.shape[0] // 2

    g = _gather_add(px, qx, src, dst)
    e = _edge_mlp(g, edge_attr, *edge_w)
    partials = _segment_sum(e, dst, zeros, n)
    x_out = _node_mlp(x, partials, va_t, vb_t, row(np_["b"][0]), np_["W"][1].T,
                      row(np_["b"][1]), np_["W"][2].T, row(np_["b"][2]),
                      np_["W"][3].T, row(np_["b"][3]),
                      row(np_["ln_w"]), row(np_["ln_b"]))
    return (x_out, e)
